# baseline (speedup 1.0000x reference)
import functools

import jax
import jax.numpy as jnp
from jax import lax
from jax.experimental import pallas as pl
from jax.experimental.pallas import tpu as pltpu
from jax.experimental.pallas import tpu_sc as plsc


def _colsum_body(x_ref, cs_ref, acc_ref):
    i = pl.program_id(0)

    @pl.when(i == 0)
    def _():
        acc_ref[...] = jnp.zeros_like(acc_ref)

    acc_ref[...] += jnp.sum(x_ref[...], axis=0, keepdims=True)

    @pl.when(i == pl.num_programs(0) - 1)
    def _():
        cs_ref[...] = acc_ref[...]


def _tc_colsum(xyflat):
    rows, lanes = xyflat.shape
    grid = 125
    blk = rows // grid
    return pl.pallas_call(
        _colsum_body,
        grid=(grid,),
        in_specs=[pl.BlockSpec((blk, lanes), lambda i: (i, 0))],
        out_specs=pl.BlockSpec((1, lanes), lambda i: (0, 0)),
        out_shape=jax.ShapeDtypeStruct((1, lanes), jnp.float32),
        scratch_shapes=[pltpu.VMEM((1, lanes), jnp.float32)],
    )(xyflat)


def _logbias_body(x_ref, bias_ref, out_ref):
    out_ref[...] = jnp.log(x_ref[...]) + bias_ref[...]


def _tc_logbias(xyflat, biaslane):
    rows, lanes = xyflat.shape
    grid = 125
    blk = rows // grid
    return pl.pallas_call(
        _logbias_body,
        grid=(grid,),
        in_specs=[
            pl.BlockSpec((blk, lanes), lambda i: (i, 0)),
            pl.BlockSpec((1, lanes), lambda i: (0, 0)),
        ],
        out_specs=pl.BlockSpec((blk, lanes), lambda i: (i, 0)),
        out_shape=jax.ShapeDtypeStruct((rows, lanes), jnp.float32),
    )(xyflat, biaslane)


_K = 10


def _make_sc_scorer(T_, B_):
    NW = 32
    NS = 16
    CW = 128
    chunks_per_w = B_ // (NW * CW)
    G = T_ // _K
    assert T_ == G * _K and G % 2 == 0 and G >= 4
    assert B_ == NW * CW * chunks_per_w

    mesh = plsc.VectorSubcoreMesh(core_axis_name="c", subcore_axis_name="s")

    @functools.partial(
        pl.kernel,
        mesh=mesh,
        compiler_params=pltpu.CompilerParams(use_tc_tiling_on_sc=False),
        out_type=[
            jax.ShapeDtypeStruct((B_,), jnp.float32),
            jax.ShapeDtypeStruct((B_,), jnp.float32),
        ],
        scratch_types=[
            [pltpu.VMEM((_K * CW,), jnp.int32)] * 2,
            [pltpu.VMEM((_K * CW,), jnp.int32)] * 2,
            [pltpu.VMEM((_K * CW,), jnp.float32)] * 2,
            [pltpu.VMEM((_K * CW,), jnp.float32)] * 2,
            pltpu.VMEM((CW,), jnp.float32),
            pltpu.VMEM((CW,), jnp.float32),
            pltpu.SemaphoreType.DMA,
            [pltpu.SemaphoreType.DMA] * 2,
        ],
    )
    def scorer(tab_hbm, idx_hbm, out0_hbm, out1_hbm,
               idxE, idxO, bufE, bufO, acc0, acc1, semI, semG):
        cid = lax.axis_index("c")
        sid = lax.axis_index("s")
        wid = sid * 2 + cid

        def wait_all(handles):
            for h in handles:
                h.wait()

        def stage(g, base, side):
            return [
                pltpu.async_copy(
                    idx_hbm.at[pl.ds((g * _K + j) * B_ + base, CW)],
                    idxE[side].at[pl.ds(j * CW, CW)], semI)
                for j in range(_K)
            ]

        def prep(side):
            for i in range(_K * CW // 16):
                v = idxE[side][pl.ds(16 * i, 16)] << 1
                idxE[side][pl.ds(16 * i, 16)] = v
                idxO[side][pl.ds(16 * i, 16)] = v | 1

        def fire(side):
            return [
                pltpu.async_copy(tab_hbm.at[idxE[side]], bufE[side],
                                 semG[side]),
                pltpu.async_copy(tab_hbm.at[idxO[side]], bufO[side],
                                 semG[side]),
            ]

        def accumulate(side):
            for i in range(CW // 16):
                s0 = bufE[side][pl.ds(16 * i, 16)]
                s1 = bufO[side][pl.ds(16 * i, 16)]
                for j in range(1, _K):
                    s0 = s0 + bufE[side][pl.ds(j * CW + 16 * i, 16)]
                    s1 = s1 + bufO[side][pl.ds(j * CW + 16 * i, 16)]
                plsc.addupdate(acc0.at[pl.ds(16 * i, 16)], s0)
                plsc.addupdate(acc1.at[pl.ds(16 * i, 16)], s1)

        def chunk_body(m, carry):
            base = (wid * chunks_per_w + m) * CW
            for i in range(CW // 16):
                z = jnp.zeros((16,), jnp.float32)
                acc0[pl.ds(16 * i, 16)] = z
                acc1[pl.ds(16 * i, 16)] = z
            wait_all(stage(0, base, 0))
            prep(0)

            def body(p, c):
                hA = fire(0)
                wait_all(stage(2 * p + 1, base, 1))
                prep(1)
                hB = fire(1)
                wait_all(hA)
                accumulate(0)
                hs = stage(2 * p + 2, base, 0)
                wait_all(hB)
                accumulate(1)
                wait_all(hs)
                prep(0)
                return c

            lax.fori_loop(0, G // 2 - 1, body, 0)
            hA = fire(0)
            wait_all(stage(G - 1, base, 1))
            prep(1)
            hB = fire(1)
            wait_all(hA)
            accumulate(0)
            wait_all(hB)
            accumulate(1)
            pltpu.sync_copy(acc0, out0_hbm.at[pl.ds(base, CW)])
            pltpu.sync_copy(acc1, out1_hbm.at[pl.ds(base, CW)])
            return carry

        lax.fori_loop(0, chunks_per_w, chunk_body, 0)

    return scorer


def kernel(input, xycounts, ycounts):
    x32 = input.astype(jnp.int32)
    T_, B_ = x32.shape
    V_ = xycounts.shape[0]

    xyflat = xycounts.reshape(V_ * 2 // 16, 16)
    cs = _tc_colsum(xyflat)
    z = jnp.sum(cs.reshape(8, 2), axis=0)
    bias_per_t = (-T_ * jnp.log(z) + jnp.log(ycounts)) / T_
    biaslane = jnp.tile(bias_per_t, 8).astype(jnp.float32).reshape(1, 16)
    logtab = _tc_logbias(xyflat, biaslane)
    tab1d = logtab.reshape(-1)

    scorer = _make_sc_scorer(T_, B_)
    o0, o1 = scorer(tab1d, x32.reshape(-1))
    return jnp.stack([o0, o1], axis=1)


# speedup vs baseline: 1.1016x; 1.1016x over previous
import functools

import jax
import jax.numpy as jnp
from jax import lax
from jax.experimental import pallas as pl
from jax.experimental.pallas import tpu as pltpu
from jax.experimental.pallas import tpu_sc as plsc


def _colsum_body(x_ref, cs_ref):
    cs_ref[...] = jnp.sum(x_ref[...], axis=0, keepdims=True)


def _tc_colsum(xyflat):
    rows, lanes = xyflat.shape
    return pl.pallas_call(
        _colsum_body,
        out_shape=jax.ShapeDtypeStruct((1, lanes), jnp.float32),
    )(xyflat)


def _logbias_body(x_ref, bias_ref, out_ref):
    out_ref[...] = jnp.log(x_ref[...]) + bias_ref[...]


def _tc_logbias(xyflat, biaslane):
    rows, lanes = xyflat.shape
    return pl.pallas_call(
        _logbias_body,
        out_shape=jax.ShapeDtypeStruct((rows, lanes), jnp.float32),
    )(xyflat, biaslane)


_K = 10


def _make_sc_scorer(T_, B_):
    NW = 32
    NS = 16
    CW = 128
    chunks_per_w = B_ // (NW * CW)
    G = T_ // _K
    assert T_ == G * _K and G % 2 == 0 and G >= 4
    assert B_ == NW * CW * chunks_per_w

    mesh = plsc.VectorSubcoreMesh(core_axis_name="c", subcore_axis_name="s")

    @functools.partial(
        pl.kernel,
        mesh=mesh,
        compiler_params=pltpu.CompilerParams(use_tc_tiling_on_sc=False),
        out_type=[
            jax.ShapeDtypeStruct((B_,), jnp.float32),
            jax.ShapeDtypeStruct((B_,), jnp.float32),
        ],
        scratch_types=[
            [pltpu.VMEM((_K * CW,), jnp.int32)] * 2,
            [pltpu.VMEM((_K * CW,), jnp.int32)] * 2,
            [pltpu.VMEM((_K * CW,), jnp.float32)] * 2,
            [pltpu.VMEM((_K * CW,), jnp.float32)] * 2,
            pltpu.VMEM((CW,), jnp.float32),
            pltpu.VMEM((CW,), jnp.float32),
            pltpu.SemaphoreType.DMA,
            [pltpu.SemaphoreType.DMA] * 2,
        ],
    )
    def scorer(tab_hbm, idx_hbm, out0_hbm, out1_hbm,
               idxE, idxO, bufE, bufO, acc0, acc1, semI, semG):
        cid = lax.axis_index("c")
        sid = lax.axis_index("s")
        wid = sid * 2 + cid

        def wait_all(handles):
            for h in handles:
                h.wait()

        def stage(g, base, side):
            return [
                pltpu.async_copy(
                    idx_hbm.at[pl.ds((g * _K + j) * B_ + base, CW)],
                    idxE[side].at[pl.ds(j * CW, CW)], semI)
                for j in range(_K)
            ]

        def prep(side):
            for i in range(_K * CW // 16):
                v = idxE[side][pl.ds(16 * i, 16)] << 1
                idxE[side][pl.ds(16 * i, 16)] = v
                idxO[side][pl.ds(16 * i, 16)] = v | 1

        def fire(side):
            return [
                pltpu.async_copy(tab_hbm.at[idxE[side]], bufE[side],
                                 semG[side]),
                pltpu.async_copy(tab_hbm.at[idxO[side]], bufO[side],
                                 semG[side]),
            ]

        def accumulate(side):
            for i in range(CW // 16):
                s0 = bufE[side][pl.ds(16 * i, 16)]
                s1 = bufO[side][pl.ds(16 * i, 16)]
                for j in range(1, _K):
                    s0 = s0 + bufE[side][pl.ds(j * CW + 16 * i, 16)]
                    s1 = s1 + bufO[side][pl.ds(j * CW + 16 * i, 16)]
                plsc.addupdate(acc0.at[pl.ds(16 * i, 16)], s0)
                plsc.addupdate(acc1.at[pl.ds(16 * i, 16)], s1)

        def chunk_body(m, carry):
            base = (wid * chunks_per_w + m) * CW
            for i in range(CW // 16):
                z = jnp.zeros((16,), jnp.float32)
                acc0[pl.ds(16 * i, 16)] = z
                acc1[pl.ds(16 * i, 16)] = z
            wait_all(stage(0, base, 0))
            prep(0)

            def body(p, c):
                hA = fire(0)
                wait_all(stage(2 * p + 1, base, 1))
                prep(1)
                hB = fire(1)
                wait_all(hA)
                accumulate(0)
                hs = stage(2 * p + 2, base, 0)
                wait_all(hB)
                accumulate(1)
                wait_all(hs)
                prep(0)
                return c

            lax.fori_loop(0, G // 2 - 1, body, 0)
            hA = fire(0)
            wait_all(stage(G - 1, base, 1))
            prep(1)
            hB = fire(1)
            wait_all(hA)
            accumulate(0)
            wait_all(hB)
            accumulate(1)
            pltpu.sync_copy(acc0, out0_hbm.at[pl.ds(base, CW)])
            pltpu.sync_copy(acc1, out1_hbm.at[pl.ds(base, CW)])
            return carry

        lax.fori_loop(0, chunks_per_w, chunk_body, 0)

    return scorer


def kernel(input, xycounts, ycounts):
    x32 = input.astype(jnp.int32)
    T_, B_ = x32.shape
    V_ = xycounts.shape[0]

    xyflat = xycounts.reshape(V_ * 2 // 128, 128)
    cs = _tc_colsum(xyflat)
    z = jnp.sum(cs.reshape(64, 2), axis=0)
    bias_per_t = (-T_ * jnp.log(z) + jnp.log(ycounts)) / T_
    biaslane = jnp.tile(bias_per_t, 64).astype(jnp.float32).reshape(1, 128)
    logtab = _tc_logbias(xyflat, biaslane)
    tab1d = logtab.reshape(-1)

    scorer = _make_sc_scorer(T_, B_)
    o0, o1 = scorer(tab1d, x32.reshape(-1))
    return jnp.stack([o0, o1], axis=1)


# speedup vs baseline: 1.1032x; 1.0014x over previous
import functools

import jax
import jax.numpy as jnp
from jax import lax
from jax.experimental import pallas as pl
from jax.experimental.pallas import tpu as pltpu
from jax.experimental.pallas import tpu_sc as plsc


def _colsum_body(x_ref, cs_ref):
    cs_ref[...] = jnp.sum(x_ref[...], axis=0, keepdims=True)


def _tc_colsum(xyflat):
    rows, lanes = xyflat.shape
    return pl.pallas_call(
        _colsum_body,
        out_shape=jax.ShapeDtypeStruct((1, lanes), jnp.float32),
    )(xyflat)


def _logbias_body(x_ref, bias_ref, out_ref):
    out_ref[...] = jnp.log(x_ref[...]) + bias_ref[...]


def _tc_logbias(xyflat, biaslane):
    rows, lanes = xyflat.shape
    return pl.pallas_call(
        _logbias_body,
        out_shape=jax.ShapeDtypeStruct((rows, lanes), jnp.float32),
    )(xyflat, biaslane)


def _flatten_body(x_ref, out_ref):
    rows, cols = x_ref.shape
    out_ref[...] = x_ref[...].reshape(rows * cols // 128, 128)


def _tc_flatten(x2d):
    rows, cols = x2d.shape
    grid = 25
    blk = rows // grid
    return pl.pallas_call(
        _flatten_body,
        grid=(grid,),
        in_specs=[pl.BlockSpec((blk, cols), lambda i: (i, 0))],
        out_specs=pl.BlockSpec((blk * cols // 128, 128), lambda i: (i, 0)),
        out_shape=jax.ShapeDtypeStruct((rows * cols // 128, 128), jnp.int32),
    )(x2d)


_K = 10


def _make_sc_scorer(T_, B_):
    NW = 32
    NS = 16
    CW = 128
    chunks_per_w = B_ // (NW * CW)
    G = T_ // _K
    assert T_ == G * _K and G % 2 == 0 and G >= 4
    assert B_ == NW * CW * chunks_per_w

    mesh = plsc.VectorSubcoreMesh(core_axis_name="c", subcore_axis_name="s")

    @functools.partial(
        pl.kernel,
        mesh=mesh,
        compiler_params=pltpu.CompilerParams(use_tc_tiling_on_sc=False),
        out_type=[
            jax.ShapeDtypeStruct((B_,), jnp.float32),
            jax.ShapeDtypeStruct((B_,), jnp.float32),
        ],
        scratch_types=[
            [pltpu.VMEM((_K * CW,), jnp.int32)] * 2,
            [pltpu.VMEM((_K * CW,), jnp.int32)] * 2,
            [pltpu.VMEM((_K * CW,), jnp.float32)] * 2,
            [pltpu.VMEM((_K * CW,), jnp.float32)] * 2,
            pltpu.VMEM((CW,), jnp.float32),
            pltpu.VMEM((CW,), jnp.float32),
            pltpu.SemaphoreType.DMA,
            [pltpu.SemaphoreType.DMA] * 2,
        ],
    )
    def scorer(tab_hbm, idx_hbm, out0_hbm, out1_hbm,
               idxE, idxO, bufE, bufO, acc0, acc1, semI, semG):
        cid = lax.axis_index("c")
        sid = lax.axis_index("s")
        wid = sid * 2 + cid

        def wait_all(handles):
            for h in handles:
                h.wait()

        def stage(g, base, side):
            return [
                pltpu.async_copy(
                    idx_hbm.at[pl.ds((g * _K + j) * B_ + base, CW)],
                    idxE[side].at[pl.ds(j * CW, CW)], semI)
                for j in range(_K)
            ]

        def prep(side):
            for i in range(_K * CW // 16):
                v = idxE[side][pl.ds(16 * i, 16)] << 1
                idxE[side][pl.ds(16 * i, 16)] = v
                idxO[side][pl.ds(16 * i, 16)] = v | 1

        def fire(side):
            return [
                pltpu.async_copy(tab_hbm.at[idxE[side]], bufE[side],
                                 semG[side]),
                pltpu.async_copy(tab_hbm.at[idxO[side]], bufO[side],
                                 semG[side]),
            ]

        def accumulate(side):
            for i in range(CW // 16):
                s0 = bufE[side][pl.ds(16 * i, 16)]
                s1 = bufO[side][pl.ds(16 * i, 16)]
                for j in range(1, _K):
                    s0 = s0 + bufE[side][pl.ds(j * CW + 16 * i, 16)]
                    s1 = s1 + bufO[side][pl.ds(j * CW + 16 * i, 16)]
                plsc.addupdate(acc0.at[pl.ds(16 * i, 16)], s0)
                plsc.addupdate(acc1.at[pl.ds(16 * i, 16)], s1)

        def chunk_body(m, carry):
            base = (wid * chunks_per_w + m) * CW
            for i in range(CW // 16):
                z = jnp.zeros((16,), jnp.float32)
                acc0[pl.ds(16 * i, 16)] = z
                acc1[pl.ds(16 * i, 16)] = z
            wait_all(stage(0, base, 0))
            prep(0)

            def body(p, c):
                hA = fire(0)
                wait_all(stage(2 * p + 1, base, 1))
                prep(1)
                hB = fire(1)
                wait_all(hA)
                accumulate(0)
                hs = stage(2 * p + 2, base, 0)
                wait_all(hB)
                accumulate(1)
                wait_all(hs)
                prep(0)
                return c

            lax.fori_loop(0, G // 2 - 1, body, 0)
            hA = fire(0)
            wait_all(stage(G - 1, base, 1))
            prep(1)
            hB = fire(1)
            wait_all(hA)
            accumulate(0)
            wait_all(hB)
            accumulate(1)
            pltpu.sync_copy(acc0, out0_hbm.at[pl.ds(base, CW)])
            pltpu.sync_copy(acc1, out1_hbm.at[pl.ds(base, CW)])
            return carry

        lax.fori_loop(0, chunks_per_w, chunk_body, 0)

    return scorer


def kernel(input, xycounts, ycounts):
    x32 = input.astype(jnp.int32)
    T_, B_ = x32.shape
    V_ = xycounts.shape[0]

    xyflat = xycounts.reshape(V_ * 2 // 128, 128)
    cs = _tc_colsum(xyflat)
    z = jnp.sum(cs.reshape(64, 2), axis=0)
    bias_per_t = (-T_ * jnp.log(z) + jnp.log(ycounts)) / T_
    biaslane = jnp.tile(bias_per_t, 64).astype(jnp.float32).reshape(1, 128)
    logtab = _tc_logbias(xyflat, biaslane)
    tab1d = logtab.reshape(-1)

    x1d = _tc_flatten(x32).reshape(-1)
    scorer = _make_sc_scorer(T_, B_)
    o0, o1 = scorer(tab1d, x1d)
    return jnp.stack([o0, o1], axis=1)
